# baseline (speedup 1.0000x reference)
import functools

import jax
import jax.numpy as jnp
from jax import lax
from jax.experimental import pallas as pl
from jax.experimental.pallas import tpu as pltpu
from jax.experimental.pallas import tpu_sc as plsc

NC = 2
NS = 16
NW = NC * NS

B = 16384
L = 50
D = 32
DW = 128
NUM_ROWS = 1000000
G = 8
ROW = 100
RPC = (G * L) // ROW
GD = 80
NGD = (G * L) // GD
IPC = G * L
BAGS_PER_WORKER = B // NW
CHUNKS = BAGS_PER_WORKER // G
NPAD = 4
ZROWS = ROW


def _sc_bag_mean(wpad, idx_flat, pat_rows):
    mesh = plsc.VectorSubcoreMesh(core_axis_name="c", subcore_axis_name="s",
                                  num_cores=NC, num_subcores=NS)

    @functools.partial(
        pl.kernel,
        mesh=mesh,
        compiler_params=pltpu.CompilerParams(use_tc_tiling_on_sc=False),
        out_type=jax.ShapeDtypeStruct((B, D), jnp.float32),
        scratch_types=[
            pltpu.VMEM((IPC,), jnp.int32),
            pltpu.VMEM((IPC,), jnp.int32),
            pltpu.VMEM((2, RPC, ROW), jnp.int32),
            pltpu.VMEM((IPC, DW), jnp.float32),
            pltpu.VMEM((IPC, DW), jnp.float32),
            pltpu.VMEM((G, DW), jnp.float32),
            pltpu.VMEM((G, DW), jnp.float32),
            pltpu.VMEM((G, D), jnp.float32),
            pltpu.VMEM((G, D), jnp.float32),
            pltpu.VMEM((ZROWS, DW), jnp.float32),
            pltpu.VMEM((2, 16), jnp.int32),
            pltpu.VMEM_SHARED((NS * 2 * G, DW), jnp.float32),
        ] + [pltpu.SemaphoreType.DMA] * 10,
    )
    def k(table_hbm, idx_hbm, pat_hbm, out_hbm,
          idx0, idx1, bag_v, rows0, rows1, sg0, sg1, st0, st1, zeros_v,
          regix, acc_sh, s_idx0, s_idx1, s_g0, s_g1, s_add0, s_add1, s_z0,
          s_z1, s_out0, s_out1):
        cid = lax.axis_index("c")
        sid = lax.axis_index("s")
        wid = sid * NC + cid

        idx_b = [idx0, idx1]
        rows_b = [rows0, rows1]
        stage128 = [sg0, sg1]
        stage32 = [st0, st1]
        s_idx = [s_idx0, s_idx1]
        s_g = [s_g0, s_g1]
        s_add = [s_add0, s_add1]
        s_z = [s_z0, s_z1]
        s_out = [s_out0, s_out1]

        def region(parity):
            return pl.ds((sid * 2 + parity) * G, G)

        def idx_desc(g, p):
            base = wid * (BAGS_PER_WORKER * L) + g * IPC
            return pltpu.make_async_copy(
                idx_hbm.at[pl.ds(base, IPC)], idx_b[p], s_idx[p])

        def gather_descs(p):
            return [pltpu.make_async_copy(
                        table_hbm.at[idx_b[p].at[pl.ds(j * GD, GD)]],
                        rows_b[p].at[pl.ds(j * GD, GD)], s_g[p])
                    for j in range(NGD)]

        def add_descs(q):
            return [pltpu.make_async_copy(
                        rows_b[q].at[pl.ds(j * ROW, ROW)],
                        acc_sh.at[bag_v.at[q].at[j]], s_add[q])
                    for j in range(RPC)]

        def zero_desc(p):
            return pltpu.make_async_copy(
                zeros_v.at[pl.ds(0, 16)], acc_sh.at[regix.at[p]], s_z[p])

        def out_desc(m, p):
            return pltpu.make_async_copy(
                stage32[p],
                out_hbm.at[pl.ds(wid * BAGS_PER_WORKER + m * G, G)],
                s_out[p])

        def fire_gathers(p):
            for j in range(NGD):
                pltpu.async_copy(table_hbm.at[idx_b[p].at[pl.ds(j * GD, GD)]],
                                 rows_b[p].at[pl.ds(j * GD, GD)], s_g[p])

        def add_phase(m, q, start_idx, npad=0):
            for d in gather_descs(q):
                d.wait()
            if start_idx:
                idx_desc(m + 2, q).start()
            zero_desc(q).start()
            for j in range(RPC):
                pltpu.async_copy(rows_b[q].at[pl.ds(j * ROW, ROW)],
                                 acc_sh.at[bag_v.at[q].at[j]], s_add[q],
                                 add=True)
            for j in range(npad):
                pltpu.async_copy(zeros_v, acc_sh.at[bag_v.at[q].at[0]],
                                 s_add[q], add=True)
            zero_desc(q).wait()
            for d in add_descs(q):
                d.wait()
            for j in range(npad):
                pltpu.make_async_copy(zeros_v, acc_sh.at[bag_v.at[q].at[0]],
                                      s_add[q]).wait()

        def read_phase(m, p, wait_out):
            if wait_out:
                out_desc(m - 2, p).wait()
            pltpu.sync_copy(acc_sh.at[region(p)], stage128[p])
            for i in range(G):
                for h in range(D // 16):
                    stage32[p][i, pl.ds(h * 16, 16)] = (
                        stage128[p][i, pl.ds(h * 16, 16)]
                        * jnp.float32(1.0 / L))
            out_desc(m, p).start()

        pltpu.sync_copy(pat_hbm.at[sid], bag_v)

        @pl.loop(0, ZROWS)
        def _(i):
            for h in range(DW // 16):
                zeros_v[i, pl.ds(h * 16, 16)] = jnp.zeros((16,), jnp.float32)

        for p in range(2):
            regix[p, pl.ds(0, 16)] = (
                lax.shift_right_logical(lax.iota(jnp.int32, 16), 1)
                + (sid * 2 + p) * G)
            idx_desc(p, p).start()

        idx_desc(0, 0).wait()
        fire_gathers(0)
        idx_desc(1, 1).wait()
        fire_gathers(1)
        add_phase(0, 0, start_idx=True)

        idx_desc(2, 0).wait()
        fire_gathers(0)
        add_phase(1, 1, start_idx=True)
        read_phase(0, 0, wait_out=False)

        idx_desc(3, 1).wait()
        fire_gathers(1)
        add_phase(2, 0, start_idx=True)
        read_phase(1, 1, wait_out=False)

        @pl.loop(4, CHUNKS - 2, step=2)
        def _(g):
            for p in (0, 1):
                gg = g + p
                q = 1 - p
                idx_desc(gg, p).wait()
                fire_gathers(p)
                add_phase(gg - 1, q, start_idx=True)
                read_phase(gg - 2, p, wait_out=True)

        idx_desc(CHUNKS - 2, 0).wait()
        fire_gathers(0)
        add_phase(CHUNKS - 3, 1, start_idx=True)
        read_phase(CHUNKS - 4, 0, wait_out=True)

        idx_desc(CHUNKS - 1, 1).wait()
        fire_gathers(1)
        add_phase(CHUNKS - 2, 0, start_idx=False)
        read_phase(CHUNKS - 3, 1, wait_out=True)

        add_phase(CHUNKS - 1, 1, start_idx=False, npad=NPAD)
        read_phase(CHUNKS - 2, 0, wait_out=True)
        read_phase(CHUNKS - 1, 1, wait_out=True)

        out_desc(CHUNKS - 2, 0).wait()
        out_desc(CHUNKS - 1, 1).wait()

    return k(wpad, idx_flat, pat_rows)


def kernel(input, weight):
    wpad = jnp.pad(weight, ((0, 0), (0, DW - D)))
    idx_flat = input.astype(jnp.int32).reshape(-1)
    local_bag = jnp.arange(G * L, dtype=jnp.int32) // L
    sidb = (jnp.arange(NS, dtype=jnp.int32) * 2)[:, None, None]
    par = jnp.arange(2, dtype=jnp.int32)[None, :, None]
    pat = (sidb + par) * G + local_bag[None, None, :]
    pat_rows = pat.reshape(NS, 2, RPC, ROW)
    return _sc_bag_mean(wpad, idx_flat, pat_rows)


# speedup vs baseline: 1.2912x; 1.2912x over previous
import functools

import jax
import jax.numpy as jnp
from jax import lax
from jax.experimental import pallas as pl
from jax.experimental.pallas import tpu as pltpu
from jax.experimental.pallas import tpu_sc as plsc

NC = 2
NS = 16
NW = NC * NS

B = 16384
L = 50
D = 32
NUM_ROWS = 1000000
G = 32
ROW = 100
RPC = (G * L) // ROW
GD = 80
NGD = (G * L) // GD
IPC = G * L
BAGS_PER_WORKER = B // NW
CHUNKS = BAGS_PER_WORKER // G
NPAD = 4


def _sc_bag_mean(weight, idx_rows, pat_rows):
    mesh = plsc.VectorSubcoreMesh(core_axis_name="c", subcore_axis_name="s",
                                  num_cores=NC, num_subcores=NS)

    @functools.partial(
        pl.kernel,
        mesh=mesh,
        compiler_params=pltpu.CompilerParams(use_tc_tiling_on_sc=False),
        out_type=jax.ShapeDtypeStruct((B, D), jnp.float32),
        scratch_types=[
            pltpu.VMEM((IPC,), jnp.int32),
            pltpu.VMEM((IPC,), jnp.int32),
            pltpu.VMEM((2, RPC, ROW), jnp.int32),
            pltpu.VMEM((G * L, D), jnp.float32),
            pltpu.VMEM((G * L, D), jnp.float32),
            pltpu.VMEM((G, D), jnp.float32),
            pltpu.VMEM((G, D), jnp.float32),
            pltpu.VMEM((ROW, D), jnp.float32),
            pltpu.VMEM_SHARED((NS * 2 * G, D), jnp.float32),
        ] + [pltpu.SemaphoreType.DMA] * 10,
    )
    def k(table1d_hbm, idx_hbm, pat_hbm, out_hbm,
          idx0, idx1, bag_v, rows0, rows1, st0, st1, zeros_v, acc_sh,
          s_idx0, s_idx1, s_g0, s_g1, s_add0, s_add1, s_z0, s_z1,
          s_out0, s_out1):
        table_hbm = table1d_hbm
        cid = lax.axis_index("c")
        sid = lax.axis_index("s")
        wid = sid * NC + cid

        idx_b = [idx0, idx1]
        rows_b = [rows0, rows1]
        stage_b = [st0, st1]
        s_idx = [s_idx0, s_idx1]
        s_g = [s_g0, s_g1]
        s_add = [s_add0, s_add1]
        s_z = [s_z0, s_z1]
        s_out = [s_out0, s_out1]

        def region(parity):
            return pl.ds((sid * 2 + parity) * G, G)

        pltpu.sync_copy(pat_hbm.at[sid], bag_v)

        @pl.loop(0, ROW)
        def _(i):
            for h in range(D // 16):
                zeros_v[i, pl.ds(h * 16, 16)] = jnp.zeros((16,), jnp.float32)

        for p in range(2):
            pltpu.sync_copy(zeros_v.at[pl.ds(0, G)], acc_sh.at[region(p)])

        hs = {}
        for p in range(2):
            base = wid * (BAGS_PER_WORKER * L) + p * IPC
            hs["idx", p] = pltpu.async_copy(
                idx_hbm.at[pl.ds(base, IPC)], idx_b[p], s_idx[p])

        def add_phase(m, npad):
            q = m % 2
            for h in hs["g", q]:
                h.wait()
            if m + 2 < CHUNKS:
                base = wid * (BAGS_PER_WORKER * L) + (m + 2) * IPC
                hs["idx", q] = pltpu.async_copy(
                    idx_hbm.at[pl.ds(base, IPC)], idx_b[q], s_idx[q])
            if ("z", q) in hs:
                hs.pop(("z", q)).wait()
            adds = []
            for j in range(RPC):
                adds.append(pltpu.async_copy(
                    rows_b[q].at[pl.ds(j * ROW, ROW)],
                    acc_sh.at[bag_v.at[q].at[j]], s_add[q], add=True))
            for j in range(npad):
                adds.append(pltpu.async_copy(
                    zeros_v, acc_sh.at[bag_v.at[q].at[0]], s_add[q],
                    add=True))
            for h in adds:
                h.wait()

        def read_phase(m):
            q = m % 2
            if ("out", q) in hs:
                hs.pop(("out", q)).wait()
            pltpu.sync_copy(acc_sh.at[region(q)], stage_b[q])
            hs["z", q] = pltpu.async_copy(
                zeros_v.at[pl.ds(0, G)], acc_sh.at[region(q)], s_z[q])

            @pl.loop(0, G)
            def _(i):
                for h in range(D // 16):
                    sl = (i, pl.ds(h * 16, 16))
                    stage_b[q][sl] = stage_b[q][sl] * jnp.float32(1.0 / L)

            hs["out", q] = pltpu.async_copy(
                stage_b[q],
                out_hbm.at[pl.ds(wid * BAGS_PER_WORKER + m * G, G)],
                s_out[q])

        for g in range(CHUNKS):
            p = g % 2
            hs["idx", p].wait()
            hs["g", p] = [
                pltpu.async_copy(table_hbm.at[idx_b[p].at[pl.ds(j * GD, GD)]],
                                 rows_b[p].at[pl.ds(j * GD, GD)], s_g[p])
                for j in range(NGD)]
            if g >= 1:
                add_phase(g - 1, npad=0)
            if g >= 2:
                read_phase(g - 2)
        add_phase(CHUNKS - 1, npad=NPAD)
        read_phase(CHUNKS - 2)
        read_phase(CHUNKS - 1)

        for q in range(2):
            hs["z", q].wait()
            hs["out", q].wait()

    return k(weight, idx_rows, pat_rows)


def kernel(input, weight):
    idx_rows = input.astype(jnp.int32).reshape(-1)
    local_bag = jnp.arange(G * L, dtype=jnp.int32) // L
    sidb = (jnp.arange(NS, dtype=jnp.int32) * 2)[:, None, None]
    par = jnp.arange(2, dtype=jnp.int32)[None, :, None]
    pat = (sidb + par) * G + local_bag[None, None, :]
    pat_rows = pat.reshape(NS, 2, RPC, ROW)
    return _sc_bag_mean(weight, idx_rows, pat_rows)
